# baseline (speedup 1.0000x reference)
import functools

import numpy as np

import jax
import jax.numpy as jnp
from jax import lax
from jax.experimental import pallas as pl
from jax.experimental.pallas import tpu as pltpu
from jax.experimental.pallas import tpu_sc as plsc

NC, NS, LANES = 2, 16, 16
NW = NC * NS
ROW = 128
KB = 24

_I = jnp.int32


def _sc_mesh():
  return plsc.VectorSubcoreMesh(core_axis_name="c", subcore_axis_name="s")


def _zero_fill(ref, nelem):
  z16 = jnp.zeros((LANES,), jnp.float32)

  def zf(zi, carry):
    ref[pl.ds(zi * _I(LANES), LANES)] = z16
    return carry

  lax.fori_loop(_I(0), _I(nelem // LANES), zf, _I(0))


def _make_deg_kernel(n, n_pad, e_rows):
  rows_per_tile = e_rows // NW
  chunks = rows_per_tile // KB

  @functools.partial(
      pl.kernel,
      out_type=jax.ShapeDtypeStruct((NW * n_pad,), jnp.float32),
      mesh=_sc_mesh(),
      compiler_params=pltpu.CompilerParams(
          use_tc_tiling_on_sc=False, needs_layout_passes=False),
      scratch_types=[
          pltpu.VMEM((n_pad,), jnp.float32),
          pltpu.VMEM((2, KB, ROW), jnp.int32),
          pltpu.SemaphoreType.DMA((2,)),
      ],
  )
  def deg_kernel(dst_hbm, out_hbm, acc_v, idx_d, sems):
    cid = lax.axis_index("c")
    sid = lax.axis_index("s")
    tile = cid * _I(NS) + sid
    ones16 = jnp.ones((LANES,), jnp.float32)
    _zero_fill(acc_v, n_pad)
    base = tile * _I(rows_per_tile)

    def issue(ci, p):
      pltpu.async_copy(dst_hbm.at[pl.ds(base + ci * _I(KB), KB)],
                       idx_d.at[p], sems.at[p])

    issue(_I(0), _I(0))

    def chunk(ci, carry):
      p = lax.rem(ci, _I(2))
      pltpu.make_async_copy(
          dst_hbm.at[pl.ds(base, KB)], idx_d.at[p], sems.at[p]).wait()

      @pl.when(ci + _I(1) < _I(chunks))
      def _():
        issue(ci + _I(1), _I(1) - p)

      @plsc.parallel_loop(_I(0), _I(KB), _I(1), unroll=8)
      def row(j):
        for k in range(ROW // LANES):
          d16 = idx_d[p, j, pl.ds(k * LANES, LANES)]
          plsc.addupdate_scatter(acc_v, [d16], ones16)

      return carry

    lax.fori_loop(_I(0), _I(chunks), chunk, _I(0))
    pltpu.sync_copy(acc_v, out_hbm.at[pl.ds(tile * _I(n_pad), n_pad)])

  return deg_kernel


def _make_agg1_kernel(n, n_pad, e_rows):
  rows_per_tile = e_rows // NW
  chunks = rows_per_tile // KB

  @functools.partial(
      pl.kernel,
      out_type=jax.ShapeDtypeStruct((NW * n_pad,), jnp.float32),
      mesh=_sc_mesh(),
      compiler_params=pltpu.CompilerParams(
          use_tc_tiling_on_sc=False, needs_layout_passes=False),
      scratch_types=[
          pltpu.VMEM((n_pad,), jnp.float32),
          pltpu.VMEM((n_pad,), jnp.float32),
          pltpu.VMEM((2, KB, ROW), jnp.int32),
          pltpu.VMEM((2, KB, ROW), jnp.int32),
          pltpu.SemaphoreType.DMA((2,)),
      ],
  )
  def agg1_kernel(u_hbm, src_hbm, dst_hbm, out_hbm,
                  acc_v, u_v, idx_s, idx_d, sems):
    cid = lax.axis_index("c")
    sid = lax.axis_index("s")
    tile = cid * _I(NS) + sid
    _zero_fill(acc_v, n_pad)
    pltpu.sync_copy(u_hbm, u_v.at[pl.ds(_I(0), n)])
    base = tile * _I(rows_per_tile)

    def issue(ci, p):
      r0 = base + ci * _I(KB)
      pltpu.async_copy(src_hbm.at[pl.ds(r0, KB)], idx_s.at[p], sems.at[p])
      pltpu.async_copy(dst_hbm.at[pl.ds(r0, KB)], idx_d.at[p], sems.at[p])

    issue(_I(0), _I(0))

    def chunk(ci, carry):
      p = lax.rem(ci, _I(2))
      pltpu.make_async_copy(
          src_hbm.at[pl.ds(base, KB)], idx_s.at[p], sems.at[p]).wait()
      pltpu.make_async_copy(
          dst_hbm.at[pl.ds(base, KB)], idx_d.at[p], sems.at[p]).wait()

      @pl.when(ci + _I(1) < _I(chunks))
      def _():
        issue(ci + _I(1), _I(1) - p)

      @plsc.parallel_loop(_I(0), _I(KB), _I(1), unroll=8)
      def row(j):
        for k in range(ROW // LANES):
          s16 = idx_s[p, j, pl.ds(k * LANES, LANES)]
          d16 = idx_d[p, j, pl.ds(k * LANES, LANES)]
          vals = plsc.load_gather(u_v, [s16])
          plsc.addupdate_scatter(acc_v, [d16], vals)

      return carry

    lax.fori_loop(_I(0), _I(chunks), chunk, _I(0))
    pltpu.sync_copy(acc_v, out_hbm.at[pl.ds(tile * _I(n_pad), n_pad)])

  return agg1_kernel


def _make_aggcol_kernel(n, n_pad, e_rows, w):
  assert w in (4, 28)

  @functools.partial(
      pl.kernel,
      out_type=jax.ShapeDtypeStruct((NW * n_pad,), jnp.float32),
      mesh=_sc_mesh(),
      compiler_params=pltpu.CompilerParams(
          use_tc_tiling_on_sc=False, needs_layout_passes=False),
      scratch_types=[
          pltpu.VMEM((n_pad,), jnp.float32),
          pltpu.VMEM((n_pad,), jnp.float32),
          pltpu.VMEM((2, KB, ROW), jnp.int32),
          pltpu.VMEM((2, KB, ROW), jnp.int32),
          pltpu.SemaphoreType.DMA((2,)),
      ],
  )
  def aggcol_kernel(ut_hbm, src_hbm, dst_hbm, out_hbm,
                    acc_v, u_v, idx_s, idx_d, sems):
    cid = lax.axis_index("c")
    sid = lax.axis_index("s")
    tile = cid * _I(NS) + sid
    if w == 4:
      col = tile // _I(8)
      part = tile - col * _I(8)
      seg = e_rows // 8
      lo = part * _I(seg)
      hi = lo + _I(seg)
    else:
      extra = (tile >= _I(28)).astype(jnp.int32)
      col = tile - _I(28) * extra
      half = e_rows // 2
      dup = jnp.logical_or(extra == 1, col < _I(4)).astype(jnp.int32)
      lo = extra * _I(half)
      hi = jnp.where(dup == 1, lo + _I(half), _I(e_rows))
    _zero_fill(acc_v, n_pad)
    pltpu.sync_copy(ut_hbm.at[col], u_v.at[pl.ds(_I(0), n)])
    nchunks = (hi - lo) // _I(KB)

    def issue(ci, p):
      r0 = lo + ci * _I(KB)
      pltpu.async_copy(src_hbm.at[pl.ds(r0, KB)], idx_s.at[p], sems.at[p])
      pltpu.async_copy(dst_hbm.at[pl.ds(r0, KB)], idx_d.at[p], sems.at[p])

    issue(_I(0), _I(0))

    def chunk(ci, carry):
      p = lax.rem(ci, _I(2))
      pltpu.make_async_copy(
          src_hbm.at[pl.ds(lo, KB)], idx_s.at[p], sems.at[p]).wait()
      pltpu.make_async_copy(
          dst_hbm.at[pl.ds(lo, KB)], idx_d.at[p], sems.at[p]).wait()

      @pl.when(ci + _I(1) < nchunks)
      def _():
        issue(ci + _I(1), _I(1) - p)

      @plsc.parallel_loop(_I(0), _I(KB), _I(1), unroll=16)
      def row(j):
        for k in range(ROW // LANES):
          s16 = idx_s[p, j, pl.ds(k * LANES, LANES)]
          d16 = idx_d[p, j, pl.ds(k * LANES, LANES)]
          vals = plsc.load_gather(u_v, [s16])
          plsc.addupdate_scatter(acc_v, [d16], vals)

      return carry

    lax.fori_loop(_I(0), nchunks, chunk, _I(0))
    pltpu.sync_copy(acc_v, out_hbm.at[pl.ds(tile * _I(n_pad), n_pad)])

  return aggcol_kernel


_RBL = 2048
_Z = np.int32(0)


def _tc1_body(degs, xt, dis_o, u0t_o):
  deg = jnp.sum(degs[...], axis=0) + 1.0
  dis = lax.rsqrt(deg)
  dis_o[...] = dis
  u0t_o[...] = xt[...] * dis[None, :]


def _tc2_body(p4, u0t, dis, w1t, b1c, st_o, u1t_o):
  psum = jnp.sum(p4[...].reshape(4, 8, p4.shape[-1]), axis=1)
  st = dis[...] * (psum + u0t[...])
  h1t = jnp.maximum(
      jnp.dot(w1t[...], st, preferred_element_type=jnp.float32) + b1c[...],
      0.0)
  st_o[...] = st
  u1t_o[...] = dis[...] * h1t


def _tc3_body(p28, u1t, dis, st, xt, w2at, w2bt, b2c, w3at, w3bt, u2_o):
  p = p28[...]
  tt = dis[...] * (jnp.concatenate([p[0:4] + p[28:32], p[4:28]], axis=0)
                   + u1t[...])
  a2 = (jnp.dot(w2at[...], tt, preferred_element_type=jnp.float32)
        + jnp.dot(w2bt[...], st[...], preferred_element_type=jnp.float32)
        + b2c[...])
  h2t = jnp.maximum(a2, 0.0)
  z = (jnp.dot(w3at[...], h2t, preferred_element_type=jnp.float32)
       + jnp.dot(w3bt[...], xt[...], preferred_element_type=jnp.float32))
  u2_o[...] = dis[...] * z


def _tc4_body(accs, u2, dis, b3, out_o):
  out_o[...] = dis[...] * (jnp.sum(accs[...], axis=0) + u2[...]) + b3[0]


def _lane_spec(rows):
  return pl.BlockSpec((rows, _RBL), lambda i: (_Z, i))


def _fix_spec(r, c):
  return pl.BlockSpec((r, c), lambda i: (_Z, _Z))


def kernel(x, edge_index, W1, b1, W2, b2, W3, b3):
  n = x.shape[0]
  e = edge_index.shape[1]
  f = x.shape[1]
  h = W1.shape[1]

  src = edge_index[0].astype(jnp.int32)
  dst = edge_index[1].astype(jnp.int32)
  step = NW * ROW * KB
  e_pad = ((e + step - 1) // step) * step
  e_rows = e_pad // ROW
  src = jnp.concatenate([src, jnp.zeros((e_pad - e,), jnp.int32)])
  dst = jnp.concatenate([dst, jnp.full((e_pad - e,), n, jnp.int32)])
  src2d = src.reshape(e_rows, ROW)
  dst2d = dst.reshape(e_rows, ROW)
  n_pad = ((n + 1 + NS * LANES - 1) // (NS * LANES)) * (NS * LANES)
  xt = x.T
  w1t = W1.T
  w2t = W2.T
  w3t = W3.T

  deg_k = _make_deg_kernel(n, n_pad, e_rows)
  agg_f = _make_aggcol_kernel(n, n_pad, e_rows, f)
  agg_h = _make_aggcol_kernel(n, n_pad, e_rows, h)
  agg_1 = _make_agg1_kernel(n, n_pad, e_rows)

  gridl = ((n + _RBL - 1) // _RBL,)

  degs = deg_k(dst2d).reshape(NW, n_pad)
  dis_flat, u0t = pl.pallas_call(
      _tc1_body,
      grid=gridl,
      in_specs=[_lane_spec(NW), _lane_spec(f)],
      out_specs=[pl.BlockSpec((_RBL,), lambda i: (i,)), _lane_spec(f)],
      out_shape=[jax.ShapeDtypeStruct((n,), jnp.float32),
                 jax.ShapeDtypeStruct((f, n), jnp.float32)],
  )(degs, xt)
  dis2 = dis_flat.reshape(1, n)

  p4 = agg_f(u0t, src2d, dst2d).reshape(NW, n_pad)
  st, u1t = pl.pallas_call(
      _tc2_body,
      grid=gridl,
      in_specs=[_lane_spec(NW), _lane_spec(f), _lane_spec(1),
                _fix_spec(h, f), _fix_spec(h, 1)],
      out_specs=[_lane_spec(f), _lane_spec(h)],
      out_shape=[jax.ShapeDtypeStruct((f, n), jnp.float32),
                 jax.ShapeDtypeStruct((h, n), jnp.float32)],
  )(p4, u0t, dis2, w1t, b1.reshape(h, 1))

  p28 = agg_h(u1t, src2d, dst2d).reshape(NW, n_pad)
  u2 = pl.pallas_call(
      _tc3_body,
      grid=gridl,
      in_specs=[_lane_spec(NW), _lane_spec(h), _lane_spec(1),
                _lane_spec(f), _lane_spec(f),
                _fix_spec(h, h), _fix_spec(h, f), _fix_spec(h, 1),
                _fix_spec(1, h), _fix_spec(1, f)],
      out_specs=_lane_spec(1),
      out_shape=jax.ShapeDtypeStruct((1, n), jnp.float32),
  )(p28, u1t, dis2, st, xt, w2t[:, :h], w2t[:, h:], b2.reshape(h, 1),
    w3t[:, :h], w3t[:, h:])

  acc2 = agg_1(u2.reshape(-1), src2d, dst2d).reshape(NW, n_pad)
  out_flat = pl.pallas_call(
      _tc4_body,
      grid=gridl,
      in_specs=[_lane_spec(NW),
                pl.BlockSpec((_RBL,), lambda i: (i,)),
                pl.BlockSpec((_RBL,), lambda i: (i,)),
                pl.BlockSpec((1,), lambda i: (_Z,))],
      out_specs=pl.BlockSpec((_RBL,), lambda i: (i,)),
      out_shape=jax.ShapeDtypeStruct((n,), jnp.float32),
  )(acc2, u2.reshape(-1), dis_flat, b3)
  return out_flat.reshape(n, 1)


# speedup vs baseline: 1.2503x; 1.2503x over previous
import functools

import numpy as np

import jax
import jax.numpy as jnp
from jax import lax
from jax.experimental import pallas as pl
from jax.experimental.pallas import tpu as pltpu
from jax.experimental.pallas import tpu_sc as plsc

NC, NS, LANES = 2, 16, 16
NW = NC * NS
ROW = 128
KB = 24

_I = jnp.int32


def _sc_mesh():
  return plsc.VectorSubcoreMesh(core_axis_name="c", subcore_axis_name="s")


def _zero_fill(ref, nelem):
  z16 = jnp.zeros((LANES,), jnp.float32)

  def zf(zi, carry):
    ref[pl.ds(zi * _I(LANES), LANES)] = z16
    return carry

  lax.fori_loop(_I(0), _I(nelem // LANES), zf, _I(0))


def _make_deg_kernel(n, n_pad, e_rows):
  rows_per_tile = e_rows // NW
  chunks = rows_per_tile // KB

  @functools.partial(
      pl.kernel,
      out_type=jax.ShapeDtypeStruct((NW * n_pad,), jnp.float32),
      mesh=_sc_mesh(),
      compiler_params=pltpu.CompilerParams(
          use_tc_tiling_on_sc=False, needs_layout_passes=False),
      scratch_types=[
          pltpu.VMEM((n_pad,), jnp.float32),
          pltpu.VMEM((2, KB, ROW), jnp.int32),
          pltpu.SemaphoreType.DMA((2,)),
      ],
  )
  def deg_kernel(dst_hbm, out_hbm, acc_v, idx_d, sems):
    cid = lax.axis_index("c")
    sid = lax.axis_index("s")
    tile = cid * _I(NS) + sid
    ones16 = jnp.ones((LANES,), jnp.float32)
    _zero_fill(acc_v, n_pad)
    base = tile * _I(rows_per_tile)

    def issue(ci, p):
      pltpu.async_copy(dst_hbm.at[pl.ds(base + ci * _I(KB), KB)],
                       idx_d.at[p], sems.at[p])

    issue(_I(0), _I(0))

    def chunk(ci, carry):
      p = lax.rem(ci, _I(2))

      @pl.when(ci + _I(1) < _I(chunks))
      def _():
        issue(ci + _I(1), _I(1) - p)

      pltpu.make_async_copy(
          dst_hbm.at[pl.ds(base, KB)], idx_d.at[p], sems.at[p]).wait()

      @plsc.parallel_loop(_I(0), _I(KB), _I(1), unroll=8)
      def row(j):
        for k in range(ROW // LANES):
          d16 = idx_d[p, j, pl.ds(k * LANES, LANES)]
          plsc.addupdate_scatter(acc_v, [d16], ones16)

      return carry

    lax.fori_loop(_I(0), _I(chunks), chunk, _I(0))
    pltpu.sync_copy(acc_v, out_hbm.at[pl.ds(tile * _I(n_pad), n_pad)])

  return deg_kernel


def _make_agg1_kernel(n, n_pad, e_rows):
  rows_per_tile = e_rows // NW
  chunks = rows_per_tile // KB

  @functools.partial(
      pl.kernel,
      out_type=jax.ShapeDtypeStruct((NW * n_pad,), jnp.float32),
      mesh=_sc_mesh(),
      compiler_params=pltpu.CompilerParams(
          use_tc_tiling_on_sc=False, needs_layout_passes=False),
      scratch_types=[
          pltpu.VMEM((n_pad,), jnp.float32),
          pltpu.VMEM((n_pad,), jnp.float32),
          pltpu.VMEM((2, KB, ROW), jnp.int32),
          pltpu.VMEM((2, KB, ROW), jnp.int32),
          pltpu.SemaphoreType.DMA((2,)),
      ],
  )
  def agg1_kernel(u_hbm, src_hbm, dst_hbm, out_hbm,
                  acc_v, u_v, idx_s, idx_d, sems):
    cid = lax.axis_index("c")
    sid = lax.axis_index("s")
    tile = cid * _I(NS) + sid
    _zero_fill(acc_v, n_pad)
    pltpu.sync_copy(u_hbm, u_v.at[pl.ds(_I(0), n)])
    base = tile * _I(rows_per_tile)

    def issue(ci, p):
      r0 = base + ci * _I(KB)
      pltpu.async_copy(src_hbm.at[pl.ds(r0, KB)], idx_s.at[p], sems.at[p])
      pltpu.async_copy(dst_hbm.at[pl.ds(r0, KB)], idx_d.at[p], sems.at[p])

    issue(_I(0), _I(0))

    def chunk(ci, carry):
      p = lax.rem(ci, _I(2))

      @pl.when(ci + _I(1) < _I(chunks))
      def _():
        issue(ci + _I(1), _I(1) - p)

      pltpu.make_async_copy(
          src_hbm.at[pl.ds(base, KB)], idx_s.at[p], sems.at[p]).wait()
      pltpu.make_async_copy(
          dst_hbm.at[pl.ds(base, KB)], idx_d.at[p], sems.at[p]).wait()

      @plsc.parallel_loop(_I(0), _I(KB), _I(1), unroll=8)
      def row(j):
        for k in range(ROW // LANES):
          s16 = idx_s[p, j, pl.ds(k * LANES, LANES)]
          d16 = idx_d[p, j, pl.ds(k * LANES, LANES)]
          vals = plsc.load_gather(u_v, [s16])
          plsc.addupdate_scatter(acc_v, [d16], vals)

      return carry

    lax.fori_loop(_I(0), _I(chunks), chunk, _I(0))
    pltpu.sync_copy(acc_v, out_hbm.at[pl.ds(tile * _I(n_pad), n_pad)])

  return agg1_kernel


def _make_aggcol_kernel(n, n_pad, e_rows, w):
  assert w in (4, 28)

  @functools.partial(
      pl.kernel,
      out_type=jax.ShapeDtypeStruct((NW * n_pad,), jnp.float32),
      mesh=_sc_mesh(),
      compiler_params=pltpu.CompilerParams(
          use_tc_tiling_on_sc=False, needs_layout_passes=False),
      scratch_types=[
          pltpu.VMEM((n_pad,), jnp.float32),
          pltpu.VMEM((n_pad,), jnp.float32),
          pltpu.VMEM((2, KB, ROW), jnp.int32),
          pltpu.VMEM((2, KB, ROW), jnp.int32),
          pltpu.SemaphoreType.DMA((2,)),
      ],
  )
  def aggcol_kernel(ut_hbm, src_hbm, dst_hbm, out_hbm,
                    acc_v, u_v, idx_s, idx_d, sems):
    cid = lax.axis_index("c")
    sid = lax.axis_index("s")
    tile = cid * _I(NS) + sid
    if w == 4:
      col = tile // _I(8)
      part = tile - col * _I(8)
      seg = e_rows // 8
      lo = part * _I(seg)
      hi = lo + _I(seg)
    else:
      extra = (tile >= _I(28)).astype(jnp.int32)
      col = tile - _I(28) * extra
      half = e_rows // 2
      dup = jnp.logical_or(extra == 1, col < _I(4)).astype(jnp.int32)
      lo = extra * _I(half)
      hi = jnp.where(dup == 1, lo + _I(half), _I(e_rows))
    _zero_fill(acc_v, n_pad)
    pltpu.sync_copy(ut_hbm.at[col], u_v.at[pl.ds(_I(0), n)])
    nchunks = (hi - lo) // _I(KB)

    def issue(ci, p):
      r0 = lo + ci * _I(KB)
      pltpu.async_copy(src_hbm.at[pl.ds(r0, KB)], idx_s.at[p], sems.at[p])
      pltpu.async_copy(dst_hbm.at[pl.ds(r0, KB)], idx_d.at[p], sems.at[p])

    issue(_I(0), _I(0))

    def chunk(ci, carry):
      p = lax.rem(ci, _I(2))

      @pl.when(ci + _I(1) < nchunks)
      def _():
        issue(ci + _I(1), _I(1) - p)

      pltpu.make_async_copy(
          src_hbm.at[pl.ds(lo, KB)], idx_s.at[p], sems.at[p]).wait()
      pltpu.make_async_copy(
          dst_hbm.at[pl.ds(lo, KB)], idx_d.at[p], sems.at[p]).wait()

      @plsc.parallel_loop(_I(0), _I(KB), _I(1), unroll=8)
      def row(j):
        for k in range(ROW // LANES):
          s16 = idx_s[p, j, pl.ds(k * LANES, LANES)]
          d16 = idx_d[p, j, pl.ds(k * LANES, LANES)]
          vals = plsc.load_gather(u_v, [s16])
          plsc.addupdate_scatter(acc_v, [d16], vals)

      return carry

    lax.fori_loop(_I(0), nchunks, chunk, _I(0))
    pltpu.sync_copy(acc_v, out_hbm.at[pl.ds(tile * _I(n_pad), n_pad)])

  return aggcol_kernel


_RBL = 2048
_Z = np.int32(0)


def _tc1_body(degs, xt, dis_o, u0t_o):
  deg = jnp.sum(degs[...], axis=0) + 1.0
  dis = lax.rsqrt(deg)
  dis_o[...] = dis
  u0t_o[...] = xt[...] * dis[None, :]


def _tc2_body(p4, u0t, dis, w1t, b1c, st_o, u1t_o):
  psum = jnp.sum(p4[...].reshape(4, 8, p4.shape[-1]), axis=1)
  st = dis[...] * (psum + u0t[...])
  h1t = jnp.maximum(
      jnp.dot(w1t[...], st, preferred_element_type=jnp.float32) + b1c[...],
      0.0)
  st_o[...] = st
  u1t_o[...] = dis[...] * h1t


def _tc3_body(p28, u1t, dis, st, xt, w2at, w2bt, b2c, w3at, w3bt, u2_o):
  p = p28[...]
  tt = dis[...] * (jnp.concatenate([p[0:4] + p[28:32], p[4:28]], axis=0)
                   + u1t[...])
  a2 = (jnp.dot(w2at[...], tt, preferred_element_type=jnp.float32)
        + jnp.dot(w2bt[...], st[...], preferred_element_type=jnp.float32)
        + b2c[...])
  h2t = jnp.maximum(a2, 0.0)
  z = (jnp.dot(w3at[...], h2t, preferred_element_type=jnp.float32)
       + jnp.dot(w3bt[...], xt[...], preferred_element_type=jnp.float32))
  u2_o[...] = dis[...] * z


def _tc4_body(accs, u2, dis, b3, out_o):
  out_o[...] = dis[...] * (jnp.sum(accs[...], axis=0) + u2[...]) + b3[0]


def _lane_spec(rows):
  return pl.BlockSpec((rows, _RBL), lambda i: (_Z, i))


def _fix_spec(r, c):
  return pl.BlockSpec((r, c), lambda i: (_Z, _Z))


def kernel(x, edge_index, W1, b1, W2, b2, W3, b3):
  n = x.shape[0]
  e = edge_index.shape[1]
  f = x.shape[1]
  h = W1.shape[1]

  src = edge_index[0].astype(jnp.int32)
  dst = edge_index[1].astype(jnp.int32)
  step = NW * ROW * KB
  e_pad = ((e + step - 1) // step) * step
  e_rows = e_pad // ROW
  src = jnp.concatenate([src, jnp.zeros((e_pad - e,), jnp.int32)])
  dst = jnp.concatenate([dst, jnp.full((e_pad - e,), n, jnp.int32)])
  src2d = src.reshape(e_rows, ROW)
  dst2d = dst.reshape(e_rows, ROW)
  n_pad = ((n + 1 + NS * LANES - 1) // (NS * LANES)) * (NS * LANES)
  xt = x.T
  w1t = W1.T
  w2t = W2.T
  w3t = W3.T

  deg_k = _make_deg_kernel(n, n_pad, e_rows)
  agg_f = _make_aggcol_kernel(n, n_pad, e_rows, f)
  agg_h = _make_aggcol_kernel(n, n_pad, e_rows, h)
  agg_1 = _make_agg1_kernel(n, n_pad, e_rows)

  gridl = ((n + _RBL - 1) // _RBL,)

  degs = deg_k(dst2d).reshape(NW, n_pad)
  dis_flat, u0t = pl.pallas_call(
      _tc1_body,
      grid=gridl,
      in_specs=[_lane_spec(NW), _lane_spec(f)],
      out_specs=[pl.BlockSpec((_RBL,), lambda i: (i,)), _lane_spec(f)],
      out_shape=[jax.ShapeDtypeStruct((n,), jnp.float32),
                 jax.ShapeDtypeStruct((f, n), jnp.float32)],
  )(degs, xt)
  dis2 = dis_flat.reshape(1, n)

  p4 = agg_f(u0t, src2d, dst2d).reshape(NW, n_pad)
  st, u1t = pl.pallas_call(
      _tc2_body,
      grid=gridl,
      in_specs=[_lane_spec(NW), _lane_spec(f), _lane_spec(1),
                _fix_spec(h, f), _fix_spec(h, 1)],
      out_specs=[_lane_spec(f), _lane_spec(h)],
      out_shape=[jax.ShapeDtypeStruct((f, n), jnp.float32),
                 jax.ShapeDtypeStruct((h, n), jnp.float32)],
  )(p4, u0t, dis2, w1t, b1.reshape(h, 1))

  p28 = agg_h(u1t, src2d, dst2d).reshape(NW, n_pad)
  u2 = pl.pallas_call(
      _tc3_body,
      grid=gridl,
      in_specs=[_lane_spec(NW), _lane_spec(h), _lane_spec(1),
                _lane_spec(f), _lane_spec(f),
                _fix_spec(h, h), _fix_spec(h, f), _fix_spec(h, 1),
                _fix_spec(1, h), _fix_spec(1, f)],
      out_specs=_lane_spec(1),
      out_shape=jax.ShapeDtypeStruct((1, n), jnp.float32),
  )(p28, u1t, dis2, st, xt, w2t[:, :h], w2t[:, h:], b2.reshape(h, 1),
    w3t[:, :h], w3t[:, h:])

  acc2 = agg_1(u2.reshape(-1), src2d, dst2d).reshape(NW, n_pad)
  out_flat = pl.pallas_call(
      _tc4_body,
      grid=gridl,
      in_specs=[_lane_spec(NW),
                pl.BlockSpec((_RBL,), lambda i: (i,)),
                pl.BlockSpec((_RBL,), lambda i: (i,)),
                pl.BlockSpec((1,), lambda i: (_Z,))],
      out_specs=pl.BlockSpec((_RBL,), lambda i: (i,)),
      out_shape=jax.ShapeDtypeStruct((n,), jnp.float32),
  )(acc2, u2.reshape(-1), dis_flat, b3)
  return out_flat.reshape(n, 1)


# speedup vs baseline: 1.3387x; 1.0706x over previous
import functools

import numpy as np

import jax
import jax.numpy as jnp
from jax import lax
from jax.experimental import pallas as pl
from jax.experimental.pallas import tpu as pltpu
from jax.experimental.pallas import tpu_sc as plsc

NC, NS, LANES = 2, 16, 16
NW = NC * NS
ROW = 128
KB = 32

_I = jnp.int32


def _sc_mesh():
  return plsc.VectorSubcoreMesh(core_axis_name="c", subcore_axis_name="s")


def _zero_fill(ref, nelem):
  z16 = jnp.zeros((LANES,), jnp.float32)

  def zf(zi, carry):
    ref[pl.ds(zi * _I(LANES), LANES)] = z16
    return carry

  lax.fori_loop(_I(0), _I(nelem // LANES), zf, _I(0))


def _make_deg_kernel(n, n_pad, e_rows):
  rows_per_tile = e_rows // NW
  chunks = rows_per_tile // KB

  @functools.partial(
      pl.kernel,
      out_type=jax.ShapeDtypeStruct((NW * n_pad,), jnp.float32),
      mesh=_sc_mesh(),
      compiler_params=pltpu.CompilerParams(
          use_tc_tiling_on_sc=False, needs_layout_passes=False),
      scratch_types=[
          pltpu.VMEM((n_pad,), jnp.float32),
          pltpu.VMEM((2, KB, ROW), jnp.int32),
          pltpu.SemaphoreType.DMA((2,)),
      ],
  )
  def deg_kernel(ei_hbm, out_hbm, acc_v, idx_d, sems):
    cid = lax.axis_index("c")
    sid = lax.axis_index("s")
    tile = cid * _I(NS) + sid
    ones16 = jnp.ones((LANES,), jnp.float32)
    _zero_fill(acc_v, n_pad)
    base = tile * _I(rows_per_tile)

    def issue(ci, p):
      pltpu.async_copy(ei_hbm.at[pl.ds(base + ci * _I(KB), KB)],
                       idx_d.at[p], sems.at[p])

    issue(_I(0), _I(0))

    def chunk(ci, carry):
      p = lax.rem(ci, _I(2))

      @pl.when(ci + _I(1) < _I(chunks))
      def _():
        issue(ci + _I(1), _I(1) - p)

      pltpu.make_async_copy(
          ei_hbm.at[pl.ds(base, KB)], idx_d.at[p], sems.at[p]).wait()

      @plsc.parallel_loop(_I(0), _I(KB), _I(1), unroll=8)
      def row(j):
        for k in range(ROW // LANES):
          v16 = idx_d[p, j, pl.ds(k * LANES, LANES)]
          d16 = lax.shift_right_logical(v16, _I(16))
          plsc.addupdate_scatter(acc_v, [d16], ones16)

      return carry

    lax.fori_loop(_I(0), _I(chunks), chunk, _I(0))
    pltpu.sync_copy(acc_v, out_hbm.at[pl.ds(tile * _I(n_pad), n_pad)])

  return deg_kernel


def _make_agg1_kernel(n, n_pad, e_rows):
  rows_per_tile = e_rows // NW
  chunks = rows_per_tile // KB

  @functools.partial(
      pl.kernel,
      out_type=jax.ShapeDtypeStruct((NW * n_pad,), jnp.float32),
      mesh=_sc_mesh(),
      compiler_params=pltpu.CompilerParams(
          use_tc_tiling_on_sc=False, needs_layout_passes=False),
      scratch_types=[
          pltpu.VMEM((n_pad,), jnp.float32),
          pltpu.VMEM((n_pad,), jnp.float32),
          pltpu.VMEM((2, KB, ROW), jnp.int32),
          pltpu.SemaphoreType.DMA((2,)),
      ],
  )
  def agg1_kernel(u_hbm, ei_hbm, out_hbm,
                  acc_v, u_v, idx_c, sems):
    cid = lax.axis_index("c")
    sid = lax.axis_index("s")
    tile = cid * _I(NS) + sid
    _zero_fill(acc_v, n_pad)
    pltpu.sync_copy(u_hbm, u_v.at[pl.ds(_I(0), n)])
    base = tile * _I(rows_per_tile)

    def issue(ci, p):
      r0 = base + ci * _I(KB)
      pltpu.async_copy(ei_hbm.at[pl.ds(r0, KB)], idx_c.at[p], sems.at[p])

    issue(_I(0), _I(0))

    def chunk(ci, carry):
      p = lax.rem(ci, _I(2))

      @pl.when(ci + _I(1) < _I(chunks))
      def _():
        issue(ci + _I(1), _I(1) - p)

      pltpu.make_async_copy(
          ei_hbm.at[pl.ds(base, KB)], idx_c.at[p], sems.at[p]).wait()

      @plsc.parallel_loop(_I(0), _I(KB), _I(1), unroll=8)
      def row(j):
        for k in range(ROW // LANES):
          v16 = idx_c[p, j, pl.ds(k * LANES, LANES)]
          s16 = lax.bitwise_and(v16, _I(0xFFFF))
          d16 = lax.shift_right_logical(v16, _I(16))
          vals = plsc.load_gather(u_v, [s16])
          plsc.addupdate_scatter(acc_v, [d16], vals)

      return carry

    lax.fori_loop(_I(0), _I(chunks), chunk, _I(0))
    pltpu.sync_copy(acc_v, out_hbm.at[pl.ds(tile * _I(n_pad), n_pad)])

  return agg1_kernel


def _make_aggcol_kernel(n, n_pad, e_rows, w):
  assert w in (4, 28)

  @functools.partial(
      pl.kernel,
      out_type=jax.ShapeDtypeStruct((NW * n_pad,), jnp.float32),
      mesh=_sc_mesh(),
      compiler_params=pltpu.CompilerParams(
          use_tc_tiling_on_sc=False, needs_layout_passes=False),
      scratch_types=[
          pltpu.VMEM((n_pad,), jnp.float32),
          pltpu.VMEM((n_pad,), jnp.float32),
          pltpu.VMEM((2, KB, ROW), jnp.int32),
          pltpu.SemaphoreType.DMA((2,)),
      ],
  )
  def aggcol_kernel(ut_hbm, ei_hbm, out_hbm,
                    acc_v, u_v, idx_c, sems):
    cid = lax.axis_index("c")
    sid = lax.axis_index("s")
    tile = cid * _I(NS) + sid
    if w == 4:
      col = tile // _I(8)
      part = tile - col * _I(8)
      seg = e_rows // 8
      lo = part * _I(seg)
      hi = lo + _I(seg)
    else:
      extra = (tile >= _I(28)).astype(jnp.int32)
      col = tile - _I(28) * extra
      half = e_rows // 2
      dup = jnp.logical_or(extra == 1, col < _I(4)).astype(jnp.int32)
      lo = extra * _I(half)
      hi = jnp.where(dup == 1, lo + _I(half), _I(e_rows))
    _zero_fill(acc_v, n_pad)
    pltpu.sync_copy(ut_hbm.at[col], u_v.at[pl.ds(_I(0), n)])
    nchunks = (hi - lo) // _I(KB)

    def issue(ci, p):
      r0 = lo + ci * _I(KB)
      pltpu.async_copy(ei_hbm.at[pl.ds(r0, KB)], idx_c.at[p], sems.at[p])

    issue(_I(0), _I(0))

    def chunk(ci, carry):
      p = lax.rem(ci, _I(2))

      @pl.when(ci + _I(1) < nchunks)
      def _():
        issue(ci + _I(1), _I(1) - p)

      pltpu.make_async_copy(
          ei_hbm.at[pl.ds(lo, KB)], idx_c.at[p], sems.at[p]).wait()

      @plsc.parallel_loop(_I(0), _I(KB), _I(1), unroll=8)
      def row(j):
        for k in range(ROW // LANES):
          v16 = idx_c[p, j, pl.ds(k * LANES, LANES)]
          s16 = lax.bitwise_and(v16, _I(0xFFFF))
          d16 = lax.shift_right_logical(v16, _I(16))
          vals = plsc.load_gather(u_v, [s16])
          plsc.addupdate_scatter(acc_v, [d16], vals)

      return carry

    lax.fori_loop(_I(0), nchunks, chunk, _I(0))
    pltpu.sync_copy(acc_v, out_hbm.at[pl.ds(tile * _I(n_pad), n_pad)])

  return aggcol_kernel


_RBL = 2048
_Z = np.int32(0)


def _tc1_body(degs, xt, dis_o, u0t_o):
  deg = jnp.sum(degs[...], axis=0) + 1.0
  dis = lax.rsqrt(deg)
  dis_o[...] = dis
  u0t_o[...] = xt[...] * dis[None, :]


def _tc2_body(p4, u0t, dis, w1t, b1c, st_o, u1t_o):
  psum = jnp.sum(p4[...].reshape(4, 8, p4.shape[-1]), axis=1)
  st = dis[...] * (psum + u0t[...])
  h1t = jnp.maximum(
      jnp.dot(w1t[...], st, preferred_element_type=jnp.float32) + b1c[...],
      0.0)
  st_o[...] = st
  u1t_o[...] = dis[...] * h1t


def _tc3_body(p28, u1t, dis, st, xt, w2at, w2bt, b2c, w3at, w3bt, u2_o):
  p = p28[...]
  tt = dis[...] * (jnp.concatenate([p[0:4] + p[28:32], p[4:28]], axis=0)
                   + u1t[...])
  a2 = (jnp.dot(w2at[...], tt, preferred_element_type=jnp.float32)
        + jnp.dot(w2bt[...], st[...], preferred_element_type=jnp.float32)
        + b2c[...])
  h2t = jnp.maximum(a2, 0.0)
  z = (jnp.dot(w3at[...], h2t, preferred_element_type=jnp.float32)
       + jnp.dot(w3bt[...], xt[...], preferred_element_type=jnp.float32))
  u2_o[...] = dis[...] * z


def _tc4_body(accs, u2, dis, b3, out_o):
  out_o[...] = dis[...] * (jnp.sum(accs[...], axis=0) + u2[...]) + b3[0]


def _lane_spec(rows):
  return pl.BlockSpec((rows, _RBL), lambda i: (_Z, i))


def _fix_spec(r, c):
  return pl.BlockSpec((r, c), lambda i: (_Z, _Z))


def kernel(x, edge_index, W1, b1, W2, b2, W3, b3):
  n = x.shape[0]
  e = edge_index.shape[1]
  f = x.shape[1]
  h = W1.shape[1]

  src = edge_index[0].astype(jnp.int32)
  dst = edge_index[1].astype(jnp.int32)
  step = NW * ROW * KB
  e_pad = ((e + step - 1) // step) * step
  e_rows = e_pad // ROW
  src = jnp.concatenate([src, jnp.zeros((e_pad - e,), jnp.int32)])
  dst = jnp.concatenate([dst, jnp.full((e_pad - e,), n, jnp.int32)])
  comb = jax.lax.bitcast_convert_type(
      (dst.astype(jnp.uint32) << 16) | src.astype(jnp.uint32), jnp.int32)
  ei2d = comb.reshape(e_rows, ROW)
  n_pad = ((n + 1 + NS * LANES - 1) // (NS * LANES)) * (NS * LANES)
  xt = x.T
  w1t = W1.T
  w2t = W2.T
  w3t = W3.T

  deg_k = _make_deg_kernel(n, n_pad, e_rows)
  agg_f = _make_aggcol_kernel(n, n_pad, e_rows, f)
  agg_h = _make_aggcol_kernel(n, n_pad, e_rows, h)
  agg_1 = _make_agg1_kernel(n, n_pad, e_rows)

  gridl = ((n + _RBL - 1) // _RBL,)

  degs = deg_k(ei2d).reshape(NW, n_pad)
  dis_flat, u0t = pl.pallas_call(
      _tc1_body,
      grid=gridl,
      in_specs=[_lane_spec(NW), _lane_spec(f)],
      out_specs=[pl.BlockSpec((_RBL,), lambda i: (i,)), _lane_spec(f)],
      out_shape=[jax.ShapeDtypeStruct((n,), jnp.float32),
                 jax.ShapeDtypeStruct((f, n), jnp.float32)],
  )(degs, xt)
  dis2 = dis_flat.reshape(1, n)

  p4 = agg_f(u0t, ei2d).reshape(NW, n_pad)
  st, u1t = pl.pallas_call(
      _tc2_body,
      grid=gridl,
      in_specs=[_lane_spec(NW), _lane_spec(f), _lane_spec(1),
                _fix_spec(h, f), _fix_spec(h, 1)],
      out_specs=[_lane_spec(f), _lane_spec(h)],
      out_shape=[jax.ShapeDtypeStruct((f, n), jnp.float32),
                 jax.ShapeDtypeStruct((h, n), jnp.float32)],
  )(p4, u0t, dis2, w1t, b1.reshape(h, 1))

  p28 = agg_h(u1t, ei2d).reshape(NW, n_pad)
  u2 = pl.pallas_call(
      _tc3_body,
      grid=gridl,
      in_specs=[_lane_spec(NW), _lane_spec(h), _lane_spec(1),
                _lane_spec(f), _lane_spec(f),
                _fix_spec(h, h), _fix_spec(h, f), _fix_spec(h, 1),
                _fix_spec(1, h), _fix_spec(1, f)],
      out_specs=_lane_spec(1),
      out_shape=jax.ShapeDtypeStruct((1, n), jnp.float32),
  )(p28, u1t, dis2, st, xt, w2t[:, :h], w2t[:, h:], b2.reshape(h, 1),
    w3t[:, :h], w3t[:, h:])

  acc2 = agg_1(u2.reshape(-1), ei2d).reshape(NW, n_pad)
  out_flat = pl.pallas_call(
      _tc4_body,
      grid=gridl,
      in_specs=[_lane_spec(NW),
                pl.BlockSpec((_RBL,), lambda i: (i,)),
                pl.BlockSpec((_RBL,), lambda i: (i,)),
                pl.BlockSpec((1,), lambda i: (_Z,))],
      out_specs=pl.BlockSpec((_RBL,), lambda i: (i,)),
      out_shape=jax.ShapeDtypeStruct((n,), jnp.float32),
  )(acc2, u2.reshape(-1), dis_flat, b3)
  return out_flat.reshape(n, 1)


# speedup vs baseline: 1.3756x; 1.0276x over previous
import functools

import numpy as np

import jax
import jax.numpy as jnp
from jax import lax
from jax.experimental import pallas as pl
from jax.experimental.pallas import tpu as pltpu
from jax.experimental.pallas import tpu_sc as plsc

NC, NS, LANES = 2, 16, 16
NW = NC * NS
ROW = 128
KB = 40

_I = jnp.int32


def _sc_mesh():
  return plsc.VectorSubcoreMesh(core_axis_name="c", subcore_axis_name="s")


def _zero_fill(ref, nelem):
  z16 = jnp.zeros((LANES,), jnp.float32)

  def zf(zi, carry):
    ref[pl.ds(zi * _I(LANES), LANES)] = z16
    return carry

  lax.fori_loop(_I(0), _I(nelem // LANES), zf, _I(0))


def _make_deg_kernel(n, n_pad, e_rows):
  rows_per_tile = e_rows // NW
  chunks = rows_per_tile // KB

  @functools.partial(
      pl.kernel,
      out_type=jax.ShapeDtypeStruct((NW * n_pad,), jnp.float32),
      mesh=_sc_mesh(),
      compiler_params=pltpu.CompilerParams(
          use_tc_tiling_on_sc=False, needs_layout_passes=False),
      scratch_types=[
          pltpu.VMEM((n_pad,), jnp.float32),
          pltpu.VMEM((2, KB, ROW), jnp.int32),
          pltpu.SemaphoreType.DMA((2,)),
      ],
  )
  def deg_kernel(ei_hbm, out_hbm, acc_v, idx_d, sems):
    cid = lax.axis_index("c")
    sid = lax.axis_index("s")
    tile = cid * _I(NS) + sid
    ones16 = jnp.ones((LANES,), jnp.float32)
    _zero_fill(acc_v, n_pad)
    base = tile * _I(rows_per_tile)

    def issue(ci, p):
      pltpu.async_copy(ei_hbm.at[pl.ds(base + ci * _I(KB), KB)],
                       idx_d.at[p], sems.at[p])

    issue(_I(0), _I(0))

    def chunk(ci, carry):
      p = lax.rem(ci, _I(2))

      @pl.when(ci + _I(1) < _I(chunks))
      def _():
        issue(ci + _I(1), _I(1) - p)

      pltpu.make_async_copy(
          ei_hbm.at[pl.ds(base, KB)], idx_d.at[p], sems.at[p]).wait()

      @plsc.parallel_loop(_I(0), _I(KB), _I(1), unroll=8)
      def row(j):
        for k in range(ROW // LANES):
          v16 = idx_d[p, j, pl.ds(k * LANES, LANES)]
          d16 = lax.shift_right_logical(v16, _I(16))
          plsc.addupdate_scatter(acc_v, [d16], ones16)

      return carry

    lax.fori_loop(_I(0), _I(chunks), chunk, _I(0))
    pltpu.sync_copy(acc_v, out_hbm.at[pl.ds(tile * _I(n_pad), n_pad)])

  return deg_kernel


def _make_agg1_kernel(n, n_pad, e_rows):
  rows_per_tile = e_rows // NW
  chunks = rows_per_tile // KB

  @functools.partial(
      pl.kernel,
      out_type=jax.ShapeDtypeStruct((NW * n_pad,), jnp.float32),
      mesh=_sc_mesh(),
      compiler_params=pltpu.CompilerParams(
          use_tc_tiling_on_sc=False, needs_layout_passes=False),
      scratch_types=[
          pltpu.VMEM((n_pad,), jnp.float32),
          pltpu.VMEM((n_pad,), jnp.float32),
          pltpu.VMEM((2, KB, ROW), jnp.int32),
          pltpu.SemaphoreType.DMA((2,)),
      ],
  )
  def agg1_kernel(u_hbm, ei_hbm, out_hbm,
                  acc_v, u_v, idx_c, sems):
    cid = lax.axis_index("c")
    sid = lax.axis_index("s")
    tile = cid * _I(NS) + sid
    _zero_fill(acc_v, n_pad)
    pltpu.sync_copy(u_hbm, u_v.at[pl.ds(_I(0), n)])
    base = tile * _I(rows_per_tile)

    def issue(ci, p):
      r0 = base + ci * _I(KB)
      pltpu.async_copy(ei_hbm.at[pl.ds(r0, KB)], idx_c.at[p], sems.at[p])

    issue(_I(0), _I(0))

    def chunk(ci, carry):
      p = lax.rem(ci, _I(2))

      @pl.when(ci + _I(1) < _I(chunks))
      def _():
        issue(ci + _I(1), _I(1) - p)

      pltpu.make_async_copy(
          ei_hbm.at[pl.ds(base, KB)], idx_c.at[p], sems.at[p]).wait()

      @plsc.parallel_loop(_I(0), _I(KB), _I(1), unroll=8)
      def row(j):
        for k in range(ROW // LANES):
          v16 = idx_c[p, j, pl.ds(k * LANES, LANES)]
          s16 = lax.bitwise_and(v16, _I(0xFFFF))
          d16 = lax.shift_right_logical(v16, _I(16))
          vals = plsc.load_gather(u_v, [s16])
          plsc.addupdate_scatter(acc_v, [d16], vals)

      return carry

    lax.fori_loop(_I(0), _I(chunks), chunk, _I(0))
    pltpu.sync_copy(acc_v, out_hbm.at[pl.ds(tile * _I(n_pad), n_pad)])

  return agg1_kernel


def _make_aggcol_kernel(n, n_pad, e_rows, w):
  assert w in (4, 28)

  @functools.partial(
      pl.kernel,
      out_type=jax.ShapeDtypeStruct((NW * n_pad,), jnp.float32),
      mesh=_sc_mesh(),
      compiler_params=pltpu.CompilerParams(
          use_tc_tiling_on_sc=False, needs_layout_passes=False),
      scratch_types=[
          pltpu.VMEM((n_pad,), jnp.float32),
          pltpu.VMEM((n_pad,), jnp.float32),
          pltpu.VMEM((2, KB, ROW), jnp.int32),
          pltpu.SemaphoreType.DMA((2,)),
      ],
  )
  def aggcol_kernel(ut_hbm, ei_hbm, out_hbm,
                    acc_v, u_v, idx_c, sems):
    cid = lax.axis_index("c")
    sid = lax.axis_index("s")
    tile = cid * _I(NS) + sid
    if w == 4:
      col = tile // _I(8)
      part = tile - col * _I(8)
      seg = e_rows // 8
      lo = part * _I(seg)
      hi = lo + _I(seg)
    else:
      extra = (tile >= _I(28)).astype(jnp.int32)
      col = tile - _I(28) * extra
      half = e_rows // 2
      dup = jnp.logical_or(extra == 1, col < _I(4)).astype(jnp.int32)
      lo = extra * _I(half)
      hi = jnp.where(dup == 1, lo + _I(half), _I(e_rows))
    _zero_fill(acc_v, n_pad)
    pltpu.sync_copy(ut_hbm.at[col], u_v.at[pl.ds(_I(0), n)])
    nchunks = (hi - lo) // _I(KB)

    def issue(ci, p):
      r0 = lo + ci * _I(KB)
      pltpu.async_copy(ei_hbm.at[pl.ds(r0, KB)], idx_c.at[p], sems.at[p])

    issue(_I(0), _I(0))

    def chunk(ci, carry):
      p = lax.rem(ci, _I(2))

      @pl.when(ci + _I(1) < nchunks)
      def _():
        issue(ci + _I(1), _I(1) - p)

      pltpu.make_async_copy(
          ei_hbm.at[pl.ds(lo, KB)], idx_c.at[p], sems.at[p]).wait()

      @plsc.parallel_loop(_I(0), _I(KB), _I(1), unroll=8)
      def row(j):
        for k in range(ROW // LANES):
          v16 = idx_c[p, j, pl.ds(k * LANES, LANES)]
          s16 = lax.bitwise_and(v16, _I(0xFFFF))
          d16 = lax.shift_right_logical(v16, _I(16))
          vals = plsc.load_gather(u_v, [s16])
          plsc.addupdate_scatter(acc_v, [d16], vals)

      return carry

    lax.fori_loop(_I(0), nchunks, chunk, _I(0))
    pltpu.sync_copy(acc_v, out_hbm.at[pl.ds(tile * _I(n_pad), n_pad)])

  return aggcol_kernel


_RBL = 2048
_Z = np.int32(0)


def _tc1_body(degs, xt, dis_o, u0t_o):
  deg = jnp.sum(degs[...], axis=0) + 1.0
  dis = lax.rsqrt(deg)
  dis_o[...] = dis
  u0t_o[...] = xt[...] * dis[None, :]


def _tc2_body(p4, u0t, dis, w1t, b1c, st_o, u1t_o):
  psum = jnp.sum(p4[...].reshape(4, 8, p4.shape[-1]), axis=1)
  st = dis[...] * (psum + u0t[...])
  h1t = jnp.maximum(
      jnp.dot(w1t[...], st, preferred_element_type=jnp.float32) + b1c[...],
      0.0)
  st_o[...] = st
  u1t_o[...] = dis[...] * h1t


def _tc3_body(p28, u1t, dis, st, xt, w2at, w2bt, b2c, w3at, w3bt, u2_o):
  p = p28[...]
  tt = dis[...] * (jnp.concatenate([p[0:4] + p[28:32], p[4:28]], axis=0)
                   + u1t[...])
  a2 = (jnp.dot(w2at[...], tt, preferred_element_type=jnp.float32)
        + jnp.dot(w2bt[...], st[...], preferred_element_type=jnp.float32)
        + b2c[...])
  h2t = jnp.maximum(a2, 0.0)
  z = (jnp.dot(w3at[...], h2t, preferred_element_type=jnp.float32)
       + jnp.dot(w3bt[...], xt[...], preferred_element_type=jnp.float32))
  u2_o[...] = dis[...] * z


def _tc4_body(accs, u2, dis, b3, out_o):
  out_o[...] = dis[...] * (jnp.sum(accs[...], axis=0) + u2[...]) + b3[0]


def _lane_spec(rows):
  return pl.BlockSpec((rows, _RBL), lambda i: (_Z, i))


def _fix_spec(r, c):
  return pl.BlockSpec((r, c), lambda i: (_Z, _Z))


def kernel(x, edge_index, W1, b1, W2, b2, W3, b3):
  n = x.shape[0]
  e = edge_index.shape[1]
  f = x.shape[1]
  h = W1.shape[1]

  src = edge_index[0].astype(jnp.int32)
  dst = edge_index[1].astype(jnp.int32)
  step = NW * ROW * KB
  e_pad = ((e + step - 1) // step) * step
  e_rows = e_pad // ROW
  src = jnp.concatenate([src, jnp.zeros((e_pad - e,), jnp.int32)])
  dst = jnp.concatenate([dst, jnp.full((e_pad - e,), n, jnp.int32)])
  comb = jax.lax.bitcast_convert_type(
      (dst.astype(jnp.uint32) << 16) | src.astype(jnp.uint32), jnp.int32)
  ei2d = comb.reshape(e_rows, ROW)
  n_pad = ((n + 1 + NS * LANES - 1) // (NS * LANES)) * (NS * LANES)
  xt = x.T
  w1t = W1.T
  w2t = W2.T
  w3t = W3.T

  deg_k = _make_deg_kernel(n, n_pad, e_rows)
  agg_f = _make_aggcol_kernel(n, n_pad, e_rows, f)
  agg_h = _make_aggcol_kernel(n, n_pad, e_rows, h)
  agg_1 = _make_agg1_kernel(n, n_pad, e_rows)

  gridl = ((n + _RBL - 1) // _RBL,)

  degs = deg_k(ei2d).reshape(NW, n_pad)
  dis_flat, u0t = pl.pallas_call(
      _tc1_body,
      grid=gridl,
      in_specs=[_lane_spec(NW), _lane_spec(f)],
      out_specs=[pl.BlockSpec((_RBL,), lambda i: (i,)), _lane_spec(f)],
      out_shape=[jax.ShapeDtypeStruct((n,), jnp.float32),
                 jax.ShapeDtypeStruct((f, n), jnp.float32)],
  )(degs, xt)
  dis2 = dis_flat.reshape(1, n)

  p4 = agg_f(u0t, ei2d).reshape(NW, n_pad)
  st, u1t = pl.pallas_call(
      _tc2_body,
      grid=gridl,
      in_specs=[_lane_spec(NW), _lane_spec(f), _lane_spec(1),
                _fix_spec(h, f), _fix_spec(h, 1)],
      out_specs=[_lane_spec(f), _lane_spec(h)],
      out_shape=[jax.ShapeDtypeStruct((f, n), jnp.float32),
                 jax.ShapeDtypeStruct((h, n), jnp.float32)],
  )(p4, u0t, dis2, w1t, b1.reshape(h, 1))

  p28 = agg_h(u1t, ei2d).reshape(NW, n_pad)
  u2 = pl.pallas_call(
      _tc3_body,
      grid=gridl,
      in_specs=[_lane_spec(NW), _lane_spec(h), _lane_spec(1),
                _lane_spec(f), _lane_spec(f),
                _fix_spec(h, h), _fix_spec(h, f), _fix_spec(h, 1),
                _fix_spec(1, h), _fix_spec(1, f)],
      out_specs=_lane_spec(1),
      out_shape=jax.ShapeDtypeStruct((1, n), jnp.float32),
  )(p28, u1t, dis2, st, xt, w2t[:, :h], w2t[:, h:], b2.reshape(h, 1),
    w3t[:, :h], w3t[:, h:])

  acc2 = agg_1(u2.reshape(-1), ei2d).reshape(NW, n_pad)
  out_flat = pl.pallas_call(
      _tc4_body,
      grid=gridl,
      in_specs=[_lane_spec(NW),
                pl.BlockSpec((_RBL,), lambda i: (i,)),
                pl.BlockSpec((_RBL,), lambda i: (i,)),
                pl.BlockSpec((1,), lambda i: (_Z,))],
      out_specs=pl.BlockSpec((_RBL,), lambda i: (i,)),
      out_shape=jax.ShapeDtypeStruct((n,), jnp.float32),
  )(acc2, u2.reshape(-1), dis_flat, b3)
  return out_flat.reshape(n, 1)
